# Initial kernel scaffold; baseline (speedup 1.0000x reference)
import functools

import jax
import jax.numpy as jnp
from jax import lax
from jax.experimental import pallas as pl
from jax.experimental.pallas import tpu as pltpu
from jax.experimental.pallas import tpu_sc as plsc

NC = 2
NS = 16
NW = NC * NS
L = 16


def _sc_gather(src_feat, tgt_feat, src_idx, dst_idx):
    N, D = src_feat.shape
    E = src_idx.shape[0]
    per_w = E // NW
    C = 80
    n_chunks = per_w // C
    assert per_w * NW == E and n_chunks * C == per_w

    mesh = plsc.VectorSubcoreMesh(core_axis_name="c", subcore_axis_name="s",
                                  num_cores=NC, num_subcores=NS)

    @functools.partial(
        pl.kernel,
        out_type=jax.ShapeDtypeStruct((E, D), jnp.float32),
        mesh=mesh,
        scratch_types=[
            pltpu.VMEM((C,), jnp.int32),
            pltpu.VMEM((C,), jnp.int32),
            pltpu.VMEM((C, D), jnp.float32),
            pltpu.VMEM((C, D), jnp.float32),
            pltpu.SemaphoreType.DMA,
            pltpu.SemaphoreType.DMA,
        ],
    )
    def k(src_hbm, tgt_hbm, si_hbm, di_hbm, out_hbm,
          si_v, di_v, s_rows, t_rows, sem1, sem2):
        wid = lax.axis_index("s") * NC + lax.axis_index("c")
        base_w = wid * per_w

        def chunk(j, carry):
            base = base_w + j * C
            pltpu.sync_copy(si_hbm.at[pl.ds(base, C)], si_v)
            pltpu.sync_copy(di_hbm.at[pl.ds(base, C)], di_v)
            cp1 = pltpu.async_copy(src_hbm.at[si_v], s_rows, sem1)
            cp2 = pltpu.async_copy(tgt_hbm.at[di_v], t_rows, sem2)
            cp1.wait()
            cp2.wait()

            def row(i, carry2):
                for jj in range(D // L):
                    sl = pl.ds(jj * L, L)
                    s_rows[i, sl] = s_rows[i, sl] + t_rows[i, sl]
                return carry2

            lax.fori_loop(0, C, row, 0)
            pltpu.sync_copy(s_rows, out_hbm.at[pl.ds(base, C)])
            return carry

        lax.fori_loop(0, n_chunks, chunk, 0)

    return k(src_feat, tgt_feat, src_idx, dst_idx)


def _sc_scatter(add_feat, dst_idx, num_nodes):
    E, D = add_feat.shape
    N = num_nodes
    per_core = E // NC
    per_w = per_core // NS
    C = 80
    n_chunks = per_w // C
    rows_per_s = N // NS
    ZR = 125
    assert n_chunks * C == per_w and rows_per_s * NS == N

    mesh = plsc.VectorSubcoreMesh(core_axis_name="c", subcore_axis_name="s",
                                  num_cores=NC, num_subcores=NS)

    @functools.partial(
        pl.kernel,
        out_type=(jax.ShapeDtypeStruct((NC * N, D), jnp.float32),
                  jax.ShapeDtypeStruct((NC * N, L), jnp.float32)),
        mesh=mesh,
        scratch_types=[
            pltpu.VMEM((C, D), jnp.float32),
            pltpu.VMEM((C,), jnp.int32),
            pltpu.VMEM((C, L), jnp.float32),
            pltpu.VMEM((ZR, D), jnp.float32),
            pltpu.VMEM((rows_per_s, L), jnp.float32),
            pltpu.VMEM_SHARED((N, D), jnp.float32),
            pltpu.VMEM_SHARED((N, L), jnp.float32),
            pltpu.SemaphoreType.DMA,
        ],
    )
    def k(add_hbm, di_hbm, sums_hbm, cnts_hbm,
          rows, di_v, ones_v, zacc, zcnt, acc, cnt_sh, sem):
        c = lax.axis_index("c")
        s = lax.axis_index("s")

        def zrow(i, carry):
            for jj in range(D // L):
                zacc[i, pl.ds(jj * L, L)] = jnp.zeros((L,), jnp.float32)
            return carry
        lax.fori_loop(0, ZR, zrow, 0)

        def zrow2(i, carry):
            zcnt[i, pl.ds(0, L)] = jnp.zeros((L,), jnp.float32)
            return carry
        lax.fori_loop(0, rows_per_s, zrow2, 0)

        def orow(i, carry):
            ones_v[i, pl.ds(0, L)] = jnp.ones((L,), jnp.float32)
            return carry
        lax.fori_loop(0, C, orow, 0)

        r0 = s * rows_per_s
        for r in range(rows_per_s // ZR):
            pltpu.sync_copy(zacc, acc.at[pl.ds(r0 + r * ZR, ZR)])
        pltpu.sync_copy(zcnt, cnt_sh.at[pl.ds(r0, rows_per_s)])
        plsc.subcore_barrier()

        base_s = c * per_core + s * per_w

        def chunk(j, carry):
            base = base_s + j * C
            pltpu.sync_copy(di_hbm.at[pl.ds(base, C)], di_v)
            pltpu.async_copy(add_hbm.at[pl.ds(base, C)], rows, sem).wait()
            pltpu.sync_copy(rows, acc.at[di_v], add=True)
            pltpu.sync_copy(ones_v, cnt_sh.at[di_v], add=True)
            return carry

        lax.fori_loop(0, n_chunks, chunk, 0)
        plsc.subcore_barrier()

        out0 = c * N + r0
        pltpu.sync_copy(acc.at[pl.ds(r0, rows_per_s)],
                        sums_hbm.at[pl.ds(out0, rows_per_s)])
        pltpu.sync_copy(cnt_sh.at[pl.ds(r0, rows_per_s)],
                        cnts_hbm.at[pl.ds(out0, rows_per_s)])

    return k(add_feat, dst_idx)


def _tc_edge(g, edge_feat, WsT, WeT, g1, b1):
    E, D = edge_feat.shape
    BE = 2000
    grid = E // BE
    assert grid * BE == E

    def body(g_ref, ef_ref, ws_ref, we_ref, g1_ref, b1_ref, add_ref, ne_ref):
        h = (jnp.dot(g_ref[...], ws_ref[...], preferred_element_type=jnp.float32)
             + jnp.dot(ef_ref[...], we_ref[...], preferred_element_type=jnp.float32))
        h = h * jax.nn.sigmoid(h)
        m = jnp.mean(h, axis=-1, keepdims=True)
        v = jnp.mean((h - m) * (h - m), axis=-1, keepdims=True)
        a = (h - m) / jnp.sqrt(v + 1e-5) * g1_ref[...] + b1_ref[...]
        add_ref[...] = a
        ne_ref[...] = ef_ref[...] + a

    return pl.pallas_call(
        body,
        grid=(grid,),
        in_specs=[
            pl.BlockSpec((BE, D), lambda i: (i, 0)),
            pl.BlockSpec((BE, D), lambda i: (i, 0)),
            pl.BlockSpec((D, D), lambda i: (0, 0)),
            pl.BlockSpec((D, D), lambda i: (0, 0)),
            pl.BlockSpec((1, D), lambda i: (0, 0)),
            pl.BlockSpec((1, D), lambda i: (0, 0)),
        ],
        out_specs=[
            pl.BlockSpec((BE, D), lambda i: (i, 0)),
            pl.BlockSpec((BE, D), lambda i: (i, 0)),
        ],
        out_shape=[
            jax.ShapeDtypeStruct((E, D), jnp.float32),
            jax.ShapeDtypeStruct((E, D), jnp.float32),
        ],
    )(g, edge_feat, WsT, WeT, g1, b1)


def _tc_node(sums, cnts, tgt_feat, WeT, WtT, g2, b2):
    N, D = tgt_feat.shape
    BN = 2000
    grid = N // BN
    nb = N // BN

    def body(s0_ref, s1_ref, c0_ref, c1_ref, t_ref, we_ref, wt_ref,
             g2_ref, b2_ref, out_ref):
        agg = s0_ref[...] + s1_ref[...]
        cnt = c0_ref[...][:, 0:1] + c1_ref[...][:, 0:1]
        mean = agg / jnp.maximum(cnt, 1.0)
        h = (jnp.dot(mean, we_ref[...], preferred_element_type=jnp.float32)
             + jnp.dot(t_ref[...], wt_ref[...], preferred_element_type=jnp.float32))
        h = h * jax.nn.sigmoid(h)
        m = jnp.mean(h, axis=-1, keepdims=True)
        v = jnp.mean((h - m) * (h - m), axis=-1, keepdims=True)
        a = (h - m) / jnp.sqrt(v + 1e-5) * g2_ref[...] + b2_ref[...]
        out_ref[...] = t_ref[...] + a

    return pl.pallas_call(
        body,
        grid=(grid,),
        in_specs=[
            pl.BlockSpec((BN, D), lambda i: (i, 0)),
            pl.BlockSpec((BN, D), lambda i: (i + nb, 0)),
            pl.BlockSpec((BN, L), lambda i: (i, 0)),
            pl.BlockSpec((BN, L), lambda i: (i + nb, 0)),
            pl.BlockSpec((BN, D), lambda i: (i, 0)),
            pl.BlockSpec((D, D), lambda i: (0, 0)),
            pl.BlockSpec((D, D), lambda i: (0, 0)),
            pl.BlockSpec((1, D), lambda i: (0, 0)),
            pl.BlockSpec((1, D), lambda i: (0, 0)),
        ],
        out_specs=pl.BlockSpec((BN, D), lambda i: (i, 0)),
        out_shape=jax.ShapeDtypeStruct((N, D), jnp.float32),
    )(sums, sums, cnts, cnts, tgt_feat, WeT, WtT, g2, b2)


def kernel(src_feat, tgt_feat, edge_feat, edge_index,
           W_s2e, W_e2e, W_e2t, W_t2t, ln1_g, ln1_b, ln2_g, ln2_b):
    N, D = src_feat.shape
    src_idx = edge_index[0]
    dst_idx = edge_index[1]

    g = _sc_gather(src_feat, tgt_feat, src_idx, dst_idx)
    add_feat, new_edge_feat = _tc_edge(
        g, edge_feat, W_s2e.T, W_e2e.T,
        ln1_g.reshape(1, D), ln1_b.reshape(1, D))
    sums, cnts = _sc_scatter(add_feat, dst_idx, N)
    new_tgt_feat = _tc_node(
        sums, cnts, tgt_feat, W_e2t.T, W_t2t.T,
        ln2_g.reshape(1, D), ln2_b.reshape(1, D))
    return (new_tgt_feat, new_edge_feat)


# speedup vs baseline: 3.4386x; 3.4386x over previous
import functools

import jax
import jax.numpy as jnp
from jax import lax
from jax.experimental import pallas as pl
from jax.experimental.pallas import tpu as pltpu
from jax.experimental.pallas import tpu_sc as plsc

NC = 2
NS = 16
NW = NC * NS
L = 16


def _sc_gather(src_feat, tgt_feat, src_idx, dst_idx):
    N, D = src_feat.shape
    E = src_idx.shape[0]
    per_w = E // NW
    C = 80
    n_chunks = per_w // C
    NWB = 10
    WBR = N // NWB
    ZR = 200
    assert per_w * NW == E and n_chunks * C == per_w

    mesh = plsc.VectorSubcoreMesh(core_axis_name="c", subcore_axis_name="s",
                                  num_cores=NC, num_subcores=NS)

    @functools.partial(
        pl.kernel,
        out_type=(jax.ShapeDtypeStruct((E, D), jnp.float32),
                  jax.ShapeDtypeStruct((NC * N, D), jnp.float32)),
        mesh=mesh,
        scratch_types=[
            pltpu.VMEM((C,), jnp.int32),
            pltpu.VMEM((C,), jnp.int32),
            pltpu.VMEM((C, D), jnp.float32),
            pltpu.VMEM((C, D), jnp.float32),
            pltpu.VMEM((ZR, D), jnp.float32),
            pltpu.VMEM_SHARED((N, D), jnp.float32),
            pltpu.SemaphoreType.DMA,
            pltpu.SemaphoreType.DMA,
        ],
    )
    def k(src_hbm, tgt_hbm, si_hbm, di_hbm, out_hbm, cnts_hbm,
          si_v, di_v, s_rows, t_rows, ones_v, cnt_sh, sem1, sem2):
        c = lax.axis_index("c")
        s = lax.axis_index("s")
        wid = s * NC + c
        base_w = wid * per_w

        def zrow(i, carry):
            for jj in range(D // L):
                ones_v[i, pl.ds(jj * L, L)] = jnp.zeros((L,), jnp.float32)
            return carry
        lax.fori_loop(0, ZR, zrow, 0)

        @pl.when(s < NWB)
        def _zero():
            r0 = s * WBR
            for r in range(WBR // ZR):
                pltpu.sync_copy(ones_v.at[pl.ds(0, ZR)],
                                cnt_sh.at[pl.ds(r0 + r * ZR, ZR)])

        def orow(i, carry):
            for jj in range(D // L):
                ones_v[i, pl.ds(jj * L, L)] = jnp.ones((L,), jnp.float32)
            return carry
        lax.fori_loop(0, C, orow, 0)
        plsc.subcore_barrier()

        def chunk(j, carry):
            base = base_w + j * C
            pltpu.sync_copy(si_hbm.at[pl.ds(base, C)], si_v)
            pltpu.sync_copy(di_hbm.at[pl.ds(base, C)], di_v)
            cp1 = pltpu.async_copy(src_hbm.at[si_v], s_rows, sem1)
            cp2 = pltpu.async_copy(tgt_hbm.at[di_v], t_rows, sem2)
            cp1.wait()
            cp2.wait()

            def row(i, carry2):
                for jj in range(D // L):
                    sl = pl.ds(jj * L, L)
                    s_rows[i, sl] = s_rows[i, sl] + t_rows[i, sl]
                return carry2

            lax.fori_loop(0, C, row, 0)
            pltpu.sync_copy(s_rows, out_hbm.at[pl.ds(base, C)])
            pltpu.sync_copy(ones_v.at[pl.ds(0, C)], cnt_sh.at[di_v], add=True)
            return carry

        lax.fori_loop(0, n_chunks, chunk, 0)
        plsc.subcore_barrier()

        @pl.when(s < NWB)
        def _writeback():
            r0 = s * WBR
            pltpu.sync_copy(cnt_sh.at[pl.ds(r0, WBR)],
                            cnts_hbm.at[pl.ds(c * N + r0, WBR)])

    return k(src_feat, tgt_feat, src_idx, dst_idx)


def _sc_scatter(add_feat, dst_idx, num_nodes):
    E, D = add_feat.shape
    N = num_nodes
    per_core = E // NC
    per_w = per_core // NS
    C = 80
    n_chunks = per_w // C
    NWB = 10
    WBR = N // NWB
    ZR = 200
    assert n_chunks * C == per_w and WBR * NWB == N and WBR % ZR == 0

    mesh = plsc.VectorSubcoreMesh(core_axis_name="c", subcore_axis_name="s",
                                  num_cores=NC, num_subcores=NS)

    @functools.partial(
        pl.kernel,
        out_type=jax.ShapeDtypeStruct((NC * N, D), jnp.float32),
        mesh=mesh,
        scratch_types=[
            pltpu.VMEM((C, D), jnp.float32),
            pltpu.VMEM((C,), jnp.int32),
            pltpu.VMEM((ZR, D), jnp.float32),
            pltpu.VMEM_SHARED((N, D), jnp.float32),
            pltpu.SemaphoreType.DMA,
        ],
    )
    def k(add_hbm, di_hbm, sums_hbm, rows, di_v, zacc, acc, sem):
        c = lax.axis_index("c")
        s = lax.axis_index("s")

        def zrow(i, carry):
            for jj in range(D // L):
                zacc[i, pl.ds(jj * L, L)] = jnp.zeros((L,), jnp.float32)
            return carry
        lax.fori_loop(0, ZR, zrow, 0)

        @pl.when(s < NWB)
        def _zero():
            r0 = s * WBR
            for r in range(WBR // ZR):
                pltpu.sync_copy(zacc, acc.at[pl.ds(r0 + r * ZR, ZR)])
        plsc.subcore_barrier()

        base_s = c * per_core + s * per_w

        def chunk(j, carry):
            base = base_s + j * C
            pltpu.sync_copy(di_hbm.at[pl.ds(base, C)], di_v)
            pltpu.async_copy(add_hbm.at[pl.ds(base, C)], rows, sem).wait()
            pltpu.sync_copy(rows, acc.at[di_v], add=True)
            return carry

        lax.fori_loop(0, n_chunks, chunk, 0)
        plsc.subcore_barrier()

        @pl.when(s < NWB)
        def _writeback():
            r0 = s * WBR
            pltpu.sync_copy(acc.at[pl.ds(r0, WBR)],
                            sums_hbm.at[pl.ds(c * N + r0, WBR)])

    return k(add_feat, dst_idx)


def _tc_edge(g, edge_feat, WsT, WeT, g1, b1):
    E, D = edge_feat.shape
    BE = 2000
    grid = E // BE
    assert grid * BE == E

    def body(g_ref, ef_ref, ws_ref, we_ref, g1_ref, b1_ref, add_ref, ne_ref):
        h = (jnp.dot(g_ref[...], ws_ref[...], preferred_element_type=jnp.float32)
             + jnp.dot(ef_ref[...], we_ref[...], preferred_element_type=jnp.float32))
        h = h * jax.nn.sigmoid(h)
        m = jnp.mean(h, axis=-1, keepdims=True)
        v = jnp.mean((h - m) * (h - m), axis=-1, keepdims=True)
        a = (h - m) / jnp.sqrt(v + 1e-5) * g1_ref[...] + b1_ref[...]
        add_ref[...] = a
        ne_ref[...] = ef_ref[...] + a

    return pl.pallas_call(
        body,
        grid=(grid,),
        in_specs=[
            pl.BlockSpec((BE, D), lambda i: (i, 0)),
            pl.BlockSpec((BE, D), lambda i: (i, 0)),
            pl.BlockSpec((D, D), lambda i: (0, 0)),
            pl.BlockSpec((D, D), lambda i: (0, 0)),
            pl.BlockSpec((1, D), lambda i: (0, 0)),
            pl.BlockSpec((1, D), lambda i: (0, 0)),
        ],
        out_specs=[
            pl.BlockSpec((BE, D), lambda i: (i, 0)),
            pl.BlockSpec((BE, D), lambda i: (i, 0)),
        ],
        out_shape=[
            jax.ShapeDtypeStruct((E, D), jnp.float32),
            jax.ShapeDtypeStruct((E, D), jnp.float32),
        ],
    )(g, edge_feat, WsT, WeT, g1, b1)


def _tc_node(sums, cnts, tgt_feat, WeT, WtT, g2, b2):
    N, D = tgt_feat.shape
    BN = 2000
    grid = N // BN
    nb = N // BN

    def body(s0_ref, s1_ref, c0_ref, c1_ref, t_ref, we_ref, wt_ref,
             g2_ref, b2_ref, out_ref):
        agg = s0_ref[...] + s1_ref[...]
        cnt = c0_ref[...][:, 0:1] + c1_ref[...][:, 0:1]
        mean = agg / jnp.maximum(cnt, 1.0)
        h = (jnp.dot(mean, we_ref[...], preferred_element_type=jnp.float32)
             + jnp.dot(t_ref[...], wt_ref[...], preferred_element_type=jnp.float32))
        h = h * jax.nn.sigmoid(h)
        m = jnp.mean(h, axis=-1, keepdims=True)
        v = jnp.mean((h - m) * (h - m), axis=-1, keepdims=True)
        a = (h - m) / jnp.sqrt(v + 1e-5) * g2_ref[...] + b2_ref[...]
        out_ref[...] = t_ref[...] + a

    return pl.pallas_call(
        body,
        grid=(grid,),
        in_specs=[
            pl.BlockSpec((BN, D), lambda i: (i, 0)),
            pl.BlockSpec((BN, D), lambda i: (i + nb, 0)),
            pl.BlockSpec((BN, D), lambda i: (i, 0)),
            pl.BlockSpec((BN, D), lambda i: (i + nb, 0)),
            pl.BlockSpec((BN, D), lambda i: (i, 0)),
            pl.BlockSpec((D, D), lambda i: (0, 0)),
            pl.BlockSpec((D, D), lambda i: (0, 0)),
            pl.BlockSpec((1, D), lambda i: (0, 0)),
            pl.BlockSpec((1, D), lambda i: (0, 0)),
        ],
        out_specs=pl.BlockSpec((BN, D), lambda i: (i, 0)),
        out_shape=jax.ShapeDtypeStruct((N, D), jnp.float32),
    )(sums, sums, cnts, cnts, tgt_feat, WeT, WtT, g2, b2)


def kernel(src_feat, tgt_feat, edge_feat, edge_index,
           W_s2e, W_e2e, W_e2t, W_t2t, ln1_g, ln1_b, ln2_g, ln2_b):
    N, D = src_feat.shape
    src_idx = edge_index[0]
    dst_idx = edge_index[1]

    g, cnts = _sc_gather(src_feat, tgt_feat, src_idx, dst_idx)
    add_feat, new_edge_feat = _tc_edge(
        g, edge_feat, W_s2e.T, W_e2e.T,
        ln1_g.reshape(1, D), ln1_b.reshape(1, D))
    sums = _sc_scatter(add_feat, dst_idx, N)
    new_tgt_feat = _tc_node(
        sums, cnts, tgt_feat, W_e2t.T, W_t2t.T,
        ln2_g.reshape(1, D), ln2_b.reshape(1, D))
    return (new_tgt_feat, new_edge_feat)
